# baseline (speedup 1.0000x reference)
import functools

import jax
import jax.numpy as jnp
from jax import lax
from jax.experimental import pallas as pl
from jax.experimental.pallas import tpu as pltpu
from jax.experimental.pallas import tpu_sc as plsc

M = 100000
D = 128
B = 16384
TOP = 4096
TEMP = 0.07

L = 16
NC, NS = 2, 16
NW = NC * NS
BPW = B // NW
TPW = TOP // NW

def _lhs_body(val_ref, vb_ref):
    x = val_ref[...]
    n = jnp.sqrt(jnp.sum(x * x, axis=1, keepdims=True))
    vb_ref[...] = (x / (n + 1e-12)).astype(jnp.bfloat16)


def _lhs(val):
    bm = 2048
    return pl.pallas_call(
        _lhs_body,
        grid=(B // bm,),
        in_specs=[pl.BlockSpec((bm, D), lambda i: (i, 0))],
        out_specs=pl.BlockSpec((bm, D), lambda i: (i, 0)),
        out_shape=jax.ShapeDtypeStruct((B, D), jnp.bfloat16),
    )(val)


@functools.cache
def _sc_kernels():
    mesh = plsc.VectorSubcoreMesh(core_axis_name="c", subcore_axis_name="s")
    sc_params = pltpu.CompilerParams(needs_layout_passes=False)

    winner_partial = functools.partial(
        pl.kernel,
        out_type=jax.ShapeDtypeStruct((NW, TOP), jnp.int32),
        mesh=mesh,
        compiler_params=sc_params,
        scratch_types=[
            pltpu.VMEM((BPW,), jnp.int32),
            pltpu.VMEM((TOP,), jnp.int32),
        ],
    )(_winner_partial_body)

    memtop_build = functools.partial(
        pl.kernel,
        out_type=jax.ShapeDtypeStruct((2 * TOP, D), jnp.float32),
        mesh=mesh,
        compiler_params=sc_params,
        scratch_types=[
            pltpu.VMEM((NW, TPW), jnp.int32),
            pltpu.VMEM((TPW,), jnp.int32),
            pltpu.VMEM((TPW,), jnp.int32),
            pltpu.VMEM((TPW, D), jnp.float32),
            pltpu.VMEM((TPW, D), jnp.float32),
            pltpu.SemaphoreType.DMA,
        ],
    )(_memtop_build_body)

    return winner_partial, memtop_build


def _winner_partial_body(idx_hbm, out_hbm, idx_v, wbuf):
    wid = lax.axis_index("s") * NC + lax.axis_index("c")
    base = wid * BPW
    pltpu.sync_copy(idx_hbm.at[pl.ds(base, BPW)], idx_v)

    neg1 = jnp.full((L,), -1, jnp.int32)
    for t in range(TOP // L):
        wbuf[pl.ds(t * L, L)] = neg1

    lane = lax.iota(jnp.int32, L)
    for k in range(BPW // L):
        iv = idx_v[pl.ds(k * L, L)]
        inb = iv < TOP
        ivc = jnp.where(inb, iv, 0)
        ival = (base + k * L) + lane
        for l in range(L):
            m = inb & (lane == l)
            plsc.store_scatter(wbuf, [ivc], ival, mask=m)

    pltpu.sync_copy(wbuf, out_hbm.at[wid])


def _memtop_build_body(mem_hbm, val_hbm, part_hbm, out_hbm, pb, gi, tpos, membuf, gbuf, sem):
    wid = lax.axis_index("s") * NC + lax.axis_index("c")
    base = wid * TPW

    pcp = pltpu.async_copy(part_hbm.at[:, pl.ds(base, TPW)], pb, sem)
    mcp = pltpu.async_copy(mem_hbm.at[pl.ds(base, TPW)], membuf, sem)
    pcp.wait()

    lane = lax.iota(jnp.int32, L)
    for c in range(TPW // L):
        acc = pb[0, pl.ds(c * L, L)]
        for w in range(1, NW):
            acc = jnp.maximum(acc, pb[w, pl.ds(c * L, L)])
        win = acc >= 0
        pos = (base + c * L) + lane
        gi[pl.ds(c * L, L)] = jnp.where(win, acc, pos)
        tpos[pl.ds(c * L, L)] = jnp.where(win, pos, TOP + pos)

    gcp = pltpu.async_copy(val_hbm.at[gi], gbuf, sem)
    mcp.wait()
    pltpu.sync_copy(membuf, out_hbm.at[pl.ds(base, TPW)])
    gcp.wait()
    pltpu.async_copy(gbuf, out_hbm.at[tpos], sem).wait()


def _matmul_body(val_ref, mt_ref, o_ref, mtb_ref):
    @pl.when(pl.program_id(0) == 0)
    def _cache_mt():
        m = mt_ref[...]
        n = jnp.sqrt(jnp.sum(m * m, axis=1, keepdims=True))
        mtb_ref[...] = (m / (n + 1e-12)).astype(jnp.bfloat16)

    o_ref[...] = lax.dot_general(
        val_ref[...],
        mtb_ref[...],
        (((1,), (1,)), ((), ())),
        preferred_element_type=jnp.float32,
    ) * (1.0 / TEMP)


def _matmul(val, mt):
    bm = 1024
    return pl.pallas_call(
        _matmul_body,
        grid=(B // bm,),
        in_specs=[
            pl.BlockSpec((bm, D), lambda i: (i, 0)),
            pl.BlockSpec((TOP, D), lambda i: (0, 0)),
        ],
        out_specs=pl.BlockSpec((bm, TOP), lambda i: (i, 0)),
        out_shape=jax.ShapeDtypeStruct((B, TOP), jnp.float32),
        scratch_shapes=[pltpu.VMEM((TOP, D), jnp.bfloat16)],
    )(val, mt)


def kernel(mem, idx, val):
    winner_partial, memtop_build = _sc_kernels()
    part = winner_partial(idx)
    vb = _lhs(val)
    mt = memtop_build(mem, val, part)
    return _matmul(vb, mt)


# speedup vs baseline: 1.0166x; 1.0166x over previous
import functools

import jax
import jax.numpy as jnp
from jax import lax
from jax.experimental import pallas as pl
from jax.experimental.pallas import tpu as pltpu
from jax.experimental.pallas import tpu_sc as plsc

M = 100000
D = 128
B = 16384
TOP = 4096
TEMP = 0.07

L = 16
NC, NS = 2, 16
NW = NC * NS
BPW = B // NW
TPW = TOP // NW

@functools.cache
def _sc_kernels():
    mesh = plsc.VectorSubcoreMesh(core_axis_name="c", subcore_axis_name="s")
    sc_params = pltpu.CompilerParams(needs_layout_passes=False)

    winner_partial = functools.partial(
        pl.kernel,
        out_type=jax.ShapeDtypeStruct((NW, TOP), jnp.int32),
        mesh=mesh,
        compiler_params=sc_params,
        scratch_types=[
            pltpu.VMEM((BPW,), jnp.int32),
            pltpu.VMEM((TOP,), jnp.int32),
        ],
    )(_winner_partial_body)

    memtop_build = functools.partial(
        pl.kernel,
        out_type=jax.ShapeDtypeStruct((2 * TOP, D), jnp.float32),
        mesh=mesh,
        compiler_params=sc_params,
        scratch_types=[
            pltpu.VMEM((NW, TPW), jnp.int32),
            pltpu.VMEM((TPW,), jnp.int32),
            pltpu.VMEM((TPW,), jnp.int32),
            pltpu.VMEM((TPW, D), jnp.float32),
            pltpu.VMEM((TPW, D), jnp.float32),
            pltpu.SemaphoreType.DMA,
        ],
    )(_memtop_build_body)

    return winner_partial, memtop_build


def _winner_partial_body(idx_hbm, out_hbm, idx_v, wbuf):
    wid = lax.axis_index("s") * NC + lax.axis_index("c")
    base = wid * BPW
    pltpu.sync_copy(idx_hbm.at[pl.ds(base, BPW)], idx_v)

    neg1 = jnp.full((L,), -1, jnp.int32)
    for t in range(TOP // L):
        wbuf[pl.ds(t * L, L)] = neg1

    lane = lax.iota(jnp.int32, L)
    for k in range(BPW // L):
        iv = idx_v[pl.ds(k * L, L)]
        inb = iv < TOP
        ivc = jnp.where(inb, iv, 0)
        ival = (base + k * L) + lane
        for l in range(L):
            m = inb & (lane == l)
            plsc.store_scatter(wbuf, [ivc], ival, mask=m)

    pltpu.sync_copy(wbuf, out_hbm.at[wid])


def _memtop_build_body(mem_hbm, val_hbm, part_hbm, out_hbm, pb, gi, tpos, membuf, gbuf, sem):
    wid = lax.axis_index("s") * NC + lax.axis_index("c")
    base = wid * TPW

    pcp = pltpu.async_copy(part_hbm.at[:, pl.ds(base, TPW)], pb, sem)
    mcp = pltpu.async_copy(mem_hbm.at[pl.ds(base, TPW)], membuf, sem)
    pcp.wait()

    lane = lax.iota(jnp.int32, L)
    for c in range(TPW // L):
        acc = pb[0, pl.ds(c * L, L)]
        for w in range(1, NW):
            acc = jnp.maximum(acc, pb[w, pl.ds(c * L, L)])
        win = acc >= 0
        pos = (base + c * L) + lane
        gi[pl.ds(c * L, L)] = jnp.where(win, acc, pos)
        tpos[pl.ds(c * L, L)] = jnp.where(win, pos, TOP + pos)

    gcp = pltpu.async_copy(val_hbm.at[gi], gbuf, sem)
    mcp.wait()
    pltpu.sync_copy(membuf, out_hbm.at[pl.ds(base, TPW)])
    gcp.wait()
    pltpu.async_copy(gbuf, out_hbm.at[tpos], sem).wait()


def _matmul_body(val_ref, mt_ref, o_ref, mtb_ref):
    @pl.when(pl.program_id(0) == 0)
    def _cache_mt():
        m = mt_ref[...]
        n = jnp.sqrt(jnp.sum(m * m, axis=1, keepdims=True))
        mtb_ref[...] = (m / (n + 1e-12)).astype(jnp.bfloat16)

    x = val_ref[...]
    n = jnp.sqrt(jnp.sum(x * x, axis=1, keepdims=True))
    vb = (x / (n + 1e-12)).astype(jnp.bfloat16)
    o_ref[...] = lax.dot_general(
        vb,
        mtb_ref[...],
        (((1,), (1,)), ((), ())),
        preferred_element_type=jnp.float32,
    ) * (1.0 / TEMP)


def _matmul(val, mt):
    bm = 1024
    return pl.pallas_call(
        _matmul_body,
        grid=(B // bm,),
        in_specs=[
            pl.BlockSpec((bm, D), lambda i: (i, 0)),
            pl.BlockSpec((TOP, D), lambda i: (0, 0)),
        ],
        out_specs=pl.BlockSpec((bm, TOP), lambda i: (i, 0)),
        out_shape=jax.ShapeDtypeStruct((B, TOP), jnp.float32),
        scratch_shapes=[pltpu.VMEM((TOP, D), jnp.bfloat16)],
    )(val, mt)


def kernel(mem, idx, val):
    winner_partial, memtop_build = _sc_kernels()
    part = winner_partial(idx)
    mt = memtop_build(mem, val, part)
    return _matmul(val, mt)
